# baseline (speedup 1.0000x reference)
import functools

import jax
import jax.numpy as jnp
from jax import lax
from jax.experimental import pallas as pl
from jax.experimental.pallas import tpu as pltpu
from jax.experimental.pallas import tpu_sc as plsc

_B, _V, _D = 16384, 1000000, 16
_NC, _NS = 2, 16
_NW = _NC * _NS
_BPW = _B // _NW
_CH = 128
_NCH = _BPW // _CH


def _sc_gather_body(u_h, p_h, n_h, t_mfu, t_mfi, t_mlu, t_mli,
                    o_mfu, o_mfp, o_mfn, o_mlu, o_mlp, o_mln,
                    iu, ip, inn, r0, r1, r2, r3, r4, r5, sem):
    wid = lax.axis_index("s") * _NC + lax.axis_index("c")
    pltpu.sync_copy(u_h.at[wid], iu)
    pltpu.sync_copy(p_h.at[wid], ip)
    pltpu.sync_copy(n_h.at[wid], inn)
    copies = []
    for k in range(_NCH):
        sl = pl.ds(k * _CH, _CH)
        copies.append(pltpu.async_copy(t_mfu.at[iu.at[k]], r0.at[sl], sem))
        copies.append(pltpu.async_copy(t_mfi.at[ip.at[k]], r1.at[sl], sem))
        copies.append(pltpu.async_copy(t_mfi.at[inn.at[k]], r2.at[sl], sem))
        copies.append(pltpu.async_copy(t_mlu.at[iu.at[k]], r3.at[sl], sem))
        copies.append(pltpu.async_copy(t_mli.at[ip.at[k]], r4.at[sl], sem))
        copies.append(pltpu.async_copy(t_mli.at[inn.at[k]], r5.at[sl], sem))
    for c in copies:
        c.wait()
    pltpu.sync_copy(r0, o_mfu.at[wid])
    pltpu.sync_copy(r1, o_mfp.at[wid])
    pltpu.sync_copy(r2, o_mfn.at[wid])
    pltpu.sync_copy(r3, o_mlu.at[wid])
    pltpu.sync_copy(r4, o_mlp.at[wid])
    pltpu.sync_copy(r5, o_mln.at[wid])


def _sc_gather(user, pos_item, neg_item, t_mfu, t_mfi, t_mlu, t_mli):
    mesh = plsc.VectorSubcoreMesh(core_axis_name="c", subcore_axis_name="s")
    row = jax.ShapeDtypeStruct((_NW, _BPW, _D), jnp.float32)
    fn = functools.partial(
        pl.kernel,
        mesh=mesh,
        out_type=[row] * 6,
        scratch_types=[
            pltpu.VMEM((_NCH, _CH), jnp.int32),
            pltpu.VMEM((_NCH, _CH), jnp.int32),
            pltpu.VMEM((_NCH, _CH), jnp.int32),
        ] + [pltpu.VMEM((_BPW, _D), jnp.float32)] * 6 + [
            pltpu.SemaphoreType.DMA,
        ],
        compiler_params=pltpu.CompilerParams(use_tc_tiling_on_sc=False),
    )(_sc_gather_body)
    outs = fn(user.reshape(_NW, _NCH, _CH), pos_item.reshape(_NW, _NCH, _CH),
              neg_item.reshape(_NW, _NCH, _CH), t_mfu, t_mfi, t_mlu, t_mli)
    return tuple(o.reshape(_B, _D) for o in outs)


_BLK = 2048


def _tc_body(mfu, mfp, mfn, mlu, mlpos, mlneg,
             w0, b0, w1, b1, w2, b2, w3, b3, dw, db, out):
    f32 = jnp.float32
    mf_u = mfu[...]
    pos_mf = jax.nn.sigmoid(mf_u * mfp[...])
    neg_mf = jax.nn.sigmoid(mf_u * mfn[...])
    ml_u = mlu[...]
    x = jnp.concatenate(
        [jnp.concatenate([ml_u, mlpos[...]], axis=1),
         jnp.concatenate([ml_u, mlneg[...]], axis=1)], axis=0)
    for w, b in ((w0, b0), (w1, b1), (w2, b2), (w3, b3)):
        x = jnp.maximum(jnp.dot(x, w[...], preferred_element_type=f32)
                        + b[...], 0.0)
    dwv = dw[...]
    dbv = db[...]
    pos_vec = jnp.concatenate([pos_mf, x[:_BLK]], axis=1)
    neg_vec = jnp.concatenate([neg_mf, x[_BLK:]], axis=1)
    pos_s = jnp.dot(pos_vec, dwv, preferred_element_type=f32) + dbv
    neg_s = jnp.dot(neg_vec, dwv, preferred_element_type=f32) + dbv
    out[...] = jnp.concatenate([pos_s, neg_s], axis=1)


def _tc_score(gathered, w0, b0, w1, b1, w2, b2, w3, b3, dense_w, dense_b):
    grid = _B // _BLK
    row_spec = pl.BlockSpec((_BLK, _D), lambda i: (i, 0))
    full = lambda a: pl.BlockSpec(a.shape, lambda i: (0,) * a.ndim)
    weights = (w0, b0.reshape(1, -1), w1, b1.reshape(1, -1),
               w2, b2.reshape(1, -1), w3, b3.reshape(1, -1),
               dense_w, dense_b.reshape(1, 1))
    return pl.pallas_call(
        _tc_body,
        grid=(grid,),
        in_specs=[row_spec] * 6 + [full(w) for w in weights],
        out_specs=pl.BlockSpec((_BLK, 2), lambda i: (i, 0)),
        out_shape=jax.ShapeDtypeStruct((_B, 2), jnp.float32),
    )(*gathered, *weights)


def kernel(user, pos_item, neg_item, mf_user_table, mf_item_table,
           mlp_user_table, mlp_item_table, W0, b0, W1, b1, W2, b2, W3, b3,
           dense_W, dense_b):
    user = user.reshape(-1).astype(jnp.int32)
    pos_item = pos_item.reshape(-1).astype(jnp.int32)
    neg_item = neg_item.reshape(-1).astype(jnp.int32)
    gathered = _sc_gather(user, pos_item, neg_item, mf_user_table,
                          mf_item_table, mlp_user_table, mlp_item_table)
    return _tc_score(gathered, W0, b0, W1, b1, W2, b2, W3, b3,
                     dense_W, dense_b)


# speedup vs baseline: 1.2666x; 1.2666x over previous
import functools

import jax
import jax.numpy as jnp
from jax import lax
from jax.experimental import pallas as pl
from jax.experimental.pallas import tpu as pltpu
from jax.experimental.pallas import tpu_sc as plsc

_B, _V, _D = 16384, 1000000, 16
_PACK = 8
_VP = _V // _PACK
_PW = _PACK * _D
_NC, _NS = 2, 16
_NW = _NC * _NS
_BPW = _B // _NW
_CH = 128
_NCH = _BPW // _CH


_PBLK = 4096
_PROWS = _PBLK // _PACK
_PGRID = -(-_V // _PBLK)
_VPAD = _PGRID * _PROWS


def _pack_body(t0, t1, t2, t3, o0, o1, o2, o3):
    for t, o in ((t0, o0), (t1, o1), (t2, o2), (t3, o3)):
        x = t[...]
        xt = jnp.swapaxes(x, 0, 1)
        o[...] = jnp.concatenate(
            [xt[s * _PROWS:(s + 1) * _PROWS, :] for s in range(_PACK)],
            axis=1)


def _tc_pack(tables_t):
    grid = (_PGRID,)
    in_spec = pl.BlockSpec((_D, _PBLK), lambda i: (0, i))
    out_spec = pl.BlockSpec((_PROWS, _PW), lambda i: (i, 0))
    out = jax.ShapeDtypeStruct((_VPAD, _PW), jnp.float32)
    return pl.pallas_call(
        _pack_body,
        grid=grid,
        in_specs=[in_spec] * 4,
        out_specs=[out_spec] * 4,
        out_shape=[out] * 4,
    )(*tables_t)


def _packed_row_ids(src, dst):
    for r in range(_NCH):
        for c in range(_CH // 16):
            sl = pl.ds(c * 16, 16)
            i = src[r, sl]
            dst[r, sl] = lax.shift_left(
                lax.shift_right_logical(i, 12), 9) + (i & 511)


def _sc_gather_body(u_h, p_h, n_h, t_mfu, t_mfi, t_mlu, t_mli,
                    o_mfu, o_mfp, o_mfn, o_mlu, o_mlp, o_mln,
                    iu, ip, inn, pu, pp, pn, b0, b1, gsem, wsem):
    wid = lax.axis_index("s") * _NC + lax.axis_index("c")
    pltpu.sync_copy(u_h.at[wid], iu)
    pltpu.sync_copy(p_h.at[wid], ip)
    pltpu.sync_copy(n_h.at[wid], inn)
    _packed_row_ids(iu, pu)
    _packed_row_ids(ip, pp)
    _packed_row_ids(inn, pn)
    steps = []
    for tab, pidx, out in ((t_mfu, pu, o_mfu), (t_mfi, pp, o_mfp),
                           (t_mfi, pn, o_mfn), (t_mlu, pu, o_mlu),
                           (t_mli, pp, o_mlp), (t_mli, pn, o_mln)):
        for k in range(_NCH):
            steps.append((tab, pidx, out, k))
    bufs = (b0, b1)
    writes = [None, None]
    for s, (tab, pidx, out, k) in enumerate(steps):
        buf = bufs[s % 2]
        if writes[s % 2] is not None:
            writes[s % 2].wait()
        pltpu.async_copy(tab.at[pidx.at[k]], buf, gsem).wait()
        w = pltpu.async_copy(buf, out.at[wid, pl.ds(k * _CH, _CH)], wsem)
        writes[s % 2] = w
    writes[0].wait()
    writes[1].wait()


def _sc_gather(user, pos_item, neg_item, packed):
    mesh = plsc.VectorSubcoreMesh(core_axis_name="c", subcore_axis_name="s")
    row = jax.ShapeDtypeStruct((_NW, _BPW, _PW), jnp.float32)
    fn = functools.partial(
        pl.kernel,
        mesh=mesh,
        out_type=[row] * 6,
        scratch_types=[pltpu.VMEM((_NCH, _CH), jnp.int32)] * 6 + [
            pltpu.VMEM((_CH, _PW), jnp.float32),
            pltpu.VMEM((_CH, _PW), jnp.float32),
            pltpu.SemaphoreType.DMA,
            pltpu.SemaphoreType.DMA,
        ],
    )(_sc_gather_body)
    outs = fn(user.reshape(_NW, _NCH, _CH), pos_item.reshape(_NW, _NCH, _CH),
              neg_item.reshape(_NW, _NCH, _CH), *packed)
    return tuple(o.reshape(_B, _PW) for o in outs)


_BLK = 2048


def _extract(packed, sub):
    e = jnp.zeros((_BLK, _D), jnp.float32)
    for k in range(_PACK):
        e = jnp.where(sub == k, packed[:, k * _D:(k + 1) * _D], e)
    return e


def _score_body(gmfu, gmfp, gmfn, gmlu, gmlp, gmln, ui, pi, ni,
                w0, b0, w1, b1, w2, b2, w3, b3, dw, db, out):
    f32 = jnp.float32
    us = ((ui[0, 0, :] >> 9) & 7).reshape(_BLK, 1)
    ps = ((pi[0, 0, :] >> 9) & 7).reshape(_BLK, 1)
    ns = ((ni[0, 0, :] >> 9) & 7).reshape(_BLK, 1)
    mf_u = _extract(gmfu[...], us)
    mf_p = _extract(gmfp[...], ps)
    mf_n = _extract(gmfn[...], ns)
    ml_u = _extract(gmlu[...], us)
    ml_p = _extract(gmlp[...], ps)
    ml_n = _extract(gmln[...], ns)
    pos_mf = jax.nn.sigmoid(mf_u * mf_p)
    neg_mf = jax.nn.sigmoid(mf_u * mf_n)
    x = jnp.concatenate(
        [jnp.concatenate([ml_u, ml_p], axis=1),
         jnp.concatenate([ml_u, ml_n], axis=1)], axis=0)
    for w, b in ((w0, b0), (w1, b1), (w2, b2), (w3, b3)):
        x = jnp.maximum(jnp.dot(x, w[...], preferred_element_type=f32)
                        + b[...], 0.0)
    dwv = dw[...]
    dbv = db[...]
    pos_vec = jnp.concatenate([pos_mf, x[:_BLK]], axis=1)
    neg_vec = jnp.concatenate([neg_mf, x[_BLK:]], axis=1)
    pos_s = jnp.dot(pos_vec, dwv, preferred_element_type=f32) + dbv
    neg_s = jnp.dot(neg_vec, dwv, preferred_element_type=f32) + dbv
    out[...] = jnp.concatenate([pos_s, neg_s], axis=1)


def _tc_score(gathered, user, pos_item, neg_item,
              w0, b0, w1, b1, w2, b2, w3, b3, dense_w, dense_b):
    grid = _B // _BLK
    row_spec = pl.BlockSpec((_BLK, _PW), lambda i: (i, 0))
    idx_spec = pl.BlockSpec((1, 1, _BLK), lambda i: (i, 0, 0))
    full = lambda a: pl.BlockSpec(a.shape, lambda i: (0,) * a.ndim)
    weights = (w0, b0.reshape(1, -1), w1, b1.reshape(1, -1),
               w2, b2.reshape(1, -1), w3, b3.reshape(1, -1),
               dense_w, dense_b.reshape(1, 1))
    idxs = tuple(a.reshape(grid, 1, _BLK) for a in (user, pos_item, neg_item))
    return pl.pallas_call(
        _score_body,
        grid=(grid,),
        in_specs=[row_spec] * 6 + [idx_spec] * 3 + [full(w) for w in weights],
        out_specs=pl.BlockSpec((_BLK, 2), lambda i: (i, 0)),
        out_shape=jax.ShapeDtypeStruct((_B, 2), jnp.float32),
    )(*gathered, *idxs, *weights)


def kernel(user, pos_item, neg_item, mf_user_table, mf_item_table,
           mlp_user_table, mlp_item_table, W0, b0, W1, b1, W2, b2, W3, b3,
           dense_W, dense_b):
    user = user.reshape(-1).astype(jnp.int32)
    pos_item = pos_item.reshape(-1).astype(jnp.int32)
    neg_item = neg_item.reshape(-1).astype(jnp.int32)
    packed = _tc_pack((mf_user_table.T, mf_item_table.T,
                       mlp_user_table.T, mlp_item_table.T))
    gathered = _sc_gather(user, pos_item, neg_item, packed)
    return _tc_score(gathered, user, pos_item, neg_item,
                     W0, b0, W1, b1, W2, b2, W3, b3, dense_W, dense_b)


# speedup vs baseline: 4.1440x; 3.2718x over previous
import functools

import jax
import jax.numpy as jnp
from jax import lax
from jax.experimental import pallas as pl
from jax.experimental.pallas import tpu as pltpu
from jax.experimental.pallas import tpu_sc as plsc

_B, _V, _D = 16384, 1000000, 16
_PACK = 8
_VP = _V // _PACK
_PW = _PACK * _D
_NC, _NS = 2, 16
_NW = _NC * _NS
_BPW = _B // _NW
_CH = 128
_NCH = _BPW // _CH


_PBLK = 4096
_PROWS = _PBLK // _PACK
_PGRID = -(-_V // _PBLK)
_VPAD = _PGRID * _PROWS


def _pack_body(t0, t1, t2, t3, o0, o1, o2, o3):
    ident = jnp.eye(_PW, dtype=jnp.float32)
    valid = _V - pl.program_id(0) * _PBLK
    col_ok = lax.broadcasted_iota(jnp.int32, (_D, _PBLK), 1) < valid
    for t, o in ((t0, o0), (t1, o1), (t2, o2), (t3, o3)):
        x = jnp.where(col_ok, t[...], 0.0)
        xcat = jnp.concatenate(
            [x[:, s * _PROWS:(s + 1) * _PROWS] for s in range(_PACK)],
            axis=0)
        o[...] = jax.lax.dot_general(
            xcat, ident, (((0,), (0,)), ((), ())),
            preferred_element_type=jnp.float32)


def _tc_pack(tables_t):
    grid = (_PGRID,)
    in_spec = pl.BlockSpec((_D, _PBLK), lambda i: (0, i))
    out_spec = pl.BlockSpec((_PROWS, _PW), lambda i: (i, 0))
    out = jax.ShapeDtypeStruct((_VPAD, _PW), jnp.float32)
    return pl.pallas_call(
        _pack_body,
        grid=grid,
        in_specs=[in_spec] * 4,
        out_specs=[out_spec] * 4,
        out_shape=[out] * 4,
    )(*tables_t)


def _packed_row_ids(src, dst):
    for r in range(_NCH):
        for c in range(_CH // 16):
            sl = pl.ds(c * 16, 16)
            i = src[r, sl]
            dst[r, sl] = lax.shift_left(
                lax.shift_right_logical(i, 12), 9) + (i & 511)


def _sc_gather_body(u_h, p_h, n_h, t_mfu, t_mfi, t_mlu, t_mli,
                    o_mfu, o_mfp, o_mfn, o_mlu, o_mlp, o_mln,
                    iu, ip, inn, pu, pp, pn, b0, b1, gsem, wsem):
    wid = lax.axis_index("s") * _NC + lax.axis_index("c")
    pltpu.sync_copy(u_h.at[wid], iu)
    pltpu.sync_copy(p_h.at[wid], ip)
    pltpu.sync_copy(n_h.at[wid], inn)
    _packed_row_ids(iu, pu)
    _packed_row_ids(ip, pp)
    _packed_row_ids(inn, pn)
    steps = []
    for tab, pidx, out in ((t_mfu, pu, o_mfu), (t_mfi, pp, o_mfp),
                           (t_mfi, pn, o_mfn), (t_mlu, pu, o_mlu),
                           (t_mli, pp, o_mlp), (t_mli, pn, o_mln)):
        for k in range(_NCH):
            steps.append((tab, pidx, out, k))
    bufs = (b0, b1)
    writes = [None, None]
    for s, (tab, pidx, out, k) in enumerate(steps):
        buf = bufs[s % 2]
        if writes[s % 2] is not None:
            writes[s % 2].wait()
        pltpu.async_copy(tab.at[pidx.at[k]], buf, gsem).wait()
        w = pltpu.async_copy(buf, out.at[wid, pl.ds(k * _CH, _CH)], wsem)
        writes[s % 2] = w
    writes[0].wait()
    writes[1].wait()


def _sc_gather(user, pos_item, neg_item, packed):
    mesh = plsc.VectorSubcoreMesh(core_axis_name="c", subcore_axis_name="s")
    row = jax.ShapeDtypeStruct((_NW, _BPW, _PW), jnp.float32)
    fn = functools.partial(
        pl.kernel,
        mesh=mesh,
        out_type=[row] * 6,
        scratch_types=[pltpu.VMEM((_NCH, _CH), jnp.int32)] * 6 + [
            pltpu.VMEM((_CH, _PW), jnp.float32),
            pltpu.VMEM((_CH, _PW), jnp.float32),
            pltpu.SemaphoreType.DMA,
            pltpu.SemaphoreType.DMA,
        ],
    )(_sc_gather_body)
    outs = fn(user.reshape(_NW, _NCH, _CH), pos_item.reshape(_NW, _NCH, _CH),
              neg_item.reshape(_NW, _NCH, _CH), *packed)
    return tuple(o.reshape(_B, _PW) for o in outs)


_BLK = 2048


def _lane_mask(sub):
    grp = lax.shift_right_logical(
        lax.broadcasted_iota(jnp.int32, (_BLK, _PW), 1), 4)
    return grp == sub


def _extract(packed, mask, fold):
    return jax.lax.dot_general(
        jnp.where(mask, packed, 0.0), fold, (((1,), (0,)), ((), ())),
        preferred_element_type=jnp.float32)


def _score_body(gmfu, gmfp, gmfn, gmlu, gmlp, gmln, ui, pi, ni,
                w0, b0, w1, b1, w2, b2, w3, b3, dw, db, out):
    f32 = jnp.float32
    us = ((ui[0, 0, :] >> 9) & 7).reshape(_BLK, 1)
    ps = ((pi[0, 0, :] >> 9) & 7).reshape(_BLK, 1)
    ns = ((ni[0, 0, :] >> 9) & 7).reshape(_BLK, 1)
    mu, mp, mn = _lane_mask(us), _lane_mask(ps), _lane_mask(ns)
    fold = jnp.where(
        (lax.broadcasted_iota(jnp.int32, (_PW, _D), 0) & 15)
        == lax.broadcasted_iota(jnp.int32, (_PW, _D), 1),
        1.0, 0.0).astype(f32)
    mf_u = _extract(gmfu[...], mu, fold)
    mf_p = _extract(gmfp[...], mp, fold)
    mf_n = _extract(gmfn[...], mn, fold)
    ml_u = _extract(gmlu[...], mu, fold)
    ml_p = _extract(gmlp[...], mp, fold)
    ml_n = _extract(gmln[...], mn, fold)
    pos_mf = jax.nn.sigmoid(mf_u * mf_p)
    neg_mf = jax.nn.sigmoid(mf_u * mf_n)
    x = jnp.concatenate(
        [jnp.concatenate([ml_u, ml_p], axis=1),
         jnp.concatenate([ml_u, ml_n], axis=1)], axis=0)
    for w, b in ((w0, b0), (w1, b1), (w2, b2), (w3, b3)):
        x = jnp.maximum(jnp.dot(x, w[...], preferred_element_type=f32)
                        + b[...], 0.0)
    dwv = dw[...]
    dbv = db[...]
    pos_vec = jnp.concatenate([pos_mf, x[:_BLK]], axis=1)
    neg_vec = jnp.concatenate([neg_mf, x[_BLK:]], axis=1)
    pos_s = jnp.dot(pos_vec, dwv, preferred_element_type=f32) + dbv
    neg_s = jnp.dot(neg_vec, dwv, preferred_element_type=f32) + dbv
    out[...] = jnp.concatenate([pos_s, neg_s], axis=1)


def _tc_score(gathered, user, pos_item, neg_item,
              w0, b0, w1, b1, w2, b2, w3, b3, dense_w, dense_b):
    grid = _B // _BLK
    row_spec = pl.BlockSpec((_BLK, _PW), lambda i: (i, 0))
    idx_spec = pl.BlockSpec((1, 1, _BLK), lambda i: (i, 0, 0))
    full = lambda a: pl.BlockSpec(a.shape, lambda i: (0,) * a.ndim)
    weights = (w0, b0.reshape(1, -1), w1, b1.reshape(1, -1),
               w2, b2.reshape(1, -1), w3, b3.reshape(1, -1),
               dense_w, dense_b.reshape(1, 1))
    idxs = tuple(a.reshape(grid, 1, _BLK) for a in (user, pos_item, neg_item))
    return pl.pallas_call(
        _score_body,
        grid=(grid,),
        in_specs=[row_spec] * 6 + [idx_spec] * 3 + [full(w) for w in weights],
        out_specs=pl.BlockSpec((_BLK, 2), lambda i: (i, 0)),
        out_shape=jax.ShapeDtypeStruct((_B, 2), jnp.float32),
    )(*gathered, *idxs, *weights)


def kernel(user, pos_item, neg_item, mf_user_table, mf_item_table,
           mlp_user_table, mlp_item_table, W0, b0, W1, b1, W2, b2, W3, b3,
           dense_W, dense_b):
    user = user.reshape(-1).astype(jnp.int32)
    pos_item = pos_item.reshape(-1).astype(jnp.int32)
    neg_item = neg_item.reshape(-1).astype(jnp.int32)
    packed = _tc_pack((mf_user_table.T, mf_item_table.T,
                       mlp_user_table.T, mlp_item_table.T))
    gathered = _sc_gather(user, pos_item, neg_item, packed)
    return _tc_score(gathered, user, pos_item, neg_item,
                     W0, b0, W1, b1, W2, b2, W3, b3, dense_W, dense_b)


# speedup vs baseline: 5.8941x; 1.4223x over previous
import functools

import jax
import jax.numpy as jnp
from jax import lax
from jax.experimental import pallas as pl
from jax.experimental.pallas import tpu as pltpu
from jax.experimental.pallas import tpu_sc as plsc

_B, _V, _D = 16384, 1000000, 16
_PACK = 8
_VP = _V // _PACK
_PW = _PACK * _D
_NC, _NS = 2, 16
_NW = _NC * _NS
_BPW = _B // _NW
_CH = 128
_NCH = _BPW // _CH


_PBLK = 16384
_PROWS = _PBLK // _PACK
_PGRID = -(-_V // _PBLK)
_VPAD = _PGRID * _PROWS
_BSH = _PBLK.bit_length() - 1
_RSH = _PROWS.bit_length() - 1


def _pack_body(t0, t1, t2, t3, o0, o1, o2, o3):
    ident = jnp.eye(_PW, dtype=jnp.float32)
    valid = _V - pl.program_id(0) * _PBLK
    col_ok = lax.broadcasted_iota(jnp.int32, (_D, _PBLK), 1) < valid
    for t, o in ((t0, o0), (t1, o1), (t2, o2), (t3, o3)):
        x = jnp.where(col_ok, t[...], 0.0)
        xcat = jnp.concatenate(
            [x[:, s * _PROWS:(s + 1) * _PROWS] for s in range(_PACK)],
            axis=0)
        o[...] = jax.lax.dot_general(
            xcat, ident, (((0,), (0,)), ((), ())),
            preferred_element_type=jnp.float32)


def _tc_pack(tables_t):
    grid = (_PGRID,)
    in_spec = pl.BlockSpec((_D, _PBLK), lambda i: (0, i))
    out_spec = pl.BlockSpec((_PROWS, _PW), lambda i: (i, 0))
    out = jax.ShapeDtypeStruct((_VPAD, _PW), jnp.float32)
    return pl.pallas_call(
        _pack_body,
        grid=grid,
        in_specs=[in_spec] * 4,
        out_specs=[out_spec] * 4,
        out_shape=[out] * 4,
    )(*tables_t)


def _packed_row_ids(src, dst):
    for r in range(_NCH):
        for c in range(_CH // 16):
            sl = pl.ds(c * 16, 16)
            i = src[r, sl]
            dst[r, sl] = lax.shift_left(
                lax.shift_right_logical(i, _BSH), _RSH) + (i & (_PROWS - 1))


def _sc_gather_body(u_h, p_h, n_h, t_mfu, t_mfi, t_mlu, t_mli,
                    o_mfu, o_mfp, o_mfn, o_mlu, o_mlp, o_mln,
                    iu, ip, inn, pu, pp, pn, b0, b1, gsem, wsem):
    wid = lax.axis_index("s") * _NC + lax.axis_index("c")
    pltpu.sync_copy(u_h.at[wid], iu)
    pltpu.sync_copy(p_h.at[wid], ip)
    pltpu.sync_copy(n_h.at[wid], inn)
    _packed_row_ids(iu, pu)
    _packed_row_ids(ip, pp)
    _packed_row_ids(inn, pn)
    steps = []
    for tab, pidx, out in ((t_mfu, pu, o_mfu), (t_mfi, pp, o_mfp),
                           (t_mfi, pn, o_mfn), (t_mlu, pu, o_mlu),
                           (t_mli, pp, o_mlp), (t_mli, pn, o_mln)):
        for k in range(_NCH):
            steps.append((tab, pidx, out, k))
    bufs = (b0, b1)
    writes = [None, None]
    for s, (tab, pidx, out, k) in enumerate(steps):
        buf = bufs[s % 2]
        if writes[s % 2] is not None:
            writes[s % 2].wait()
        pltpu.async_copy(tab.at[pidx.at[k]], buf, gsem).wait()
        w = pltpu.async_copy(buf, out.at[wid, pl.ds(k * _CH, _CH)], wsem)
        writes[s % 2] = w
    writes[0].wait()
    writes[1].wait()


def _sc_gather(user, pos_item, neg_item, packed):
    mesh = plsc.VectorSubcoreMesh(core_axis_name="c", subcore_axis_name="s")
    row = jax.ShapeDtypeStruct((_NW, _BPW, _PW), jnp.float32)
    fn = functools.partial(
        pl.kernel,
        mesh=mesh,
        out_type=[row] * 6,
        scratch_types=[pltpu.VMEM((_NCH, _CH), jnp.int32)] * 6 + [
            pltpu.VMEM((_CH, _PW), jnp.float32),
            pltpu.VMEM((_CH, _PW), jnp.float32),
            pltpu.SemaphoreType.DMA,
            pltpu.SemaphoreType.DMA,
        ],
    )(_sc_gather_body)
    outs = fn(user.reshape(_NW, _NCH, _CH), pos_item.reshape(_NW, _NCH, _CH),
              neg_item.reshape(_NW, _NCH, _CH), *packed)
    return tuple(o.reshape(_B, _PW) for o in outs)


_BLK = 2048


def _lane_mask(sub):
    grp = lax.shift_right_logical(
        lax.broadcasted_iota(jnp.int32, (_BLK, _PW), 1), 4)
    return grp == sub


def _extract(packed, mask, fold):
    return jax.lax.dot_general(
        jnp.where(mask, packed, 0.0), fold, (((1,), (0,)), ((), ())),
        preferred_element_type=jnp.float32)


def _score_body(gmfu, gmfp, gmfn, gmlu, gmlp, gmln, ui, pi, ni,
                w0, b0, w1, b1, w2, b2, w3, b3, dw, db, out):
    f32 = jnp.float32
    us = ((ui[0, 0, :] >> _RSH) & 7).reshape(_BLK, 1)
    ps = ((pi[0, 0, :] >> _RSH) & 7).reshape(_BLK, 1)
    ns = ((ni[0, 0, :] >> _RSH) & 7).reshape(_BLK, 1)
    mu, mp, mn = _lane_mask(us), _lane_mask(ps), _lane_mask(ns)
    fold = jnp.where(
        (lax.broadcasted_iota(jnp.int32, (_PW, _D), 0) & 15)
        == lax.broadcasted_iota(jnp.int32, (_PW, _D), 1),
        1.0, 0.0).astype(f32)
    mf_u = _extract(gmfu[...], mu, fold)
    mf_p = _extract(gmfp[...], mp, fold)
    mf_n = _extract(gmfn[...], mn, fold)
    ml_u = _extract(gmlu[...], mu, fold)
    ml_p = _extract(gmlp[...], mp, fold)
    ml_n = _extract(gmln[...], mn, fold)
    pos_mf = jax.nn.sigmoid(mf_u * mf_p)
    neg_mf = jax.nn.sigmoid(mf_u * mf_n)
    x = jnp.concatenate(
        [jnp.concatenate([ml_u, ml_p], axis=1),
         jnp.concatenate([ml_u, ml_n], axis=1)], axis=0)
    for w, b in ((w0, b0), (w1, b1), (w2, b2), (w3, b3)):
        x = jnp.maximum(jnp.dot(x, w[...], preferred_element_type=f32)
                        + b[...], 0.0)
    dwv = dw[...]
    dbv = db[...]
    pos_vec = jnp.concatenate([pos_mf, x[:_BLK]], axis=1)
    neg_vec = jnp.concatenate([neg_mf, x[_BLK:]], axis=1)
    pos_s = jnp.dot(pos_vec, dwv, preferred_element_type=f32) + dbv
    neg_s = jnp.dot(neg_vec, dwv, preferred_element_type=f32) + dbv
    out[...] = jnp.concatenate([pos_s, neg_s], axis=1)


def _tc_score(gathered, user, pos_item, neg_item,
              w0, b0, w1, b1, w2, b2, w3, b3, dense_w, dense_b):
    grid = _B // _BLK
    row_spec = pl.BlockSpec((_BLK, _PW), lambda i: (i, 0))
    idx_spec = pl.BlockSpec((1, 1, _BLK), lambda i: (i, 0, 0))
    full = lambda a: pl.BlockSpec(a.shape, lambda i: (0,) * a.ndim)
    weights = (w0, b0.reshape(1, -1), w1, b1.reshape(1, -1),
               w2, b2.reshape(1, -1), w3, b3.reshape(1, -1),
               dense_w, dense_b.reshape(1, 1))
    idxs = tuple(a.reshape(grid, 1, _BLK) for a in (user, pos_item, neg_item))
    return pl.pallas_call(
        _score_body,
        grid=(grid,),
        in_specs=[row_spec] * 6 + [idx_spec] * 3 + [full(w) for w in weights],
        out_specs=pl.BlockSpec((_BLK, 2), lambda i: (i, 0)),
        out_shape=jax.ShapeDtypeStruct((_B, 2), jnp.float32),
    )(*gathered, *idxs, *weights)


def kernel(user, pos_item, neg_item, mf_user_table, mf_item_table,
           mlp_user_table, mlp_item_table, W0, b0, W1, b1, W2, b2, W3, b3,
           dense_W, dense_b):
    user = user.reshape(-1).astype(jnp.int32)
    pos_item = pos_item.reshape(-1).astype(jnp.int32)
    neg_item = neg_item.reshape(-1).astype(jnp.int32)
    packed = _tc_pack((mf_user_table.T, mf_item_table.T,
                       mlp_user_table.T, mlp_item_table.T))
    gathered = _sc_gather(user, pos_item, neg_item, packed)
    return _tc_score(gathered, user, pos_item, neg_item,
                     W0, b0, W1, b1, W2, b2, W3, b3, dense_W, dense_b)


# speedup vs baseline: 6.1020x; 1.0353x over previous
import functools

import jax
import jax.numpy as jnp
from jax import lax
from jax.experimental import pallas as pl
from jax.experimental.pallas import tpu as pltpu
from jax.experimental.pallas import tpu_sc as plsc

_B, _V, _D = 16384, 1000000, 16
_PACK = 8
_PW = _PACK * _D
_NC, _NS = 2, 16
_NW = _NC * _NS
_BPW = _B // _NW
_CH = 128
_NCH = _BPW // _CH

_PBLK = 16384
_PROWS = _PBLK // _PACK
_PGRID = -(-_V // _PBLK)
_VPAD = _PGRID * _PROWS
_VROWS = _VPAD * _PACK
_BSH = _PBLK.bit_length() - 1
_RSH = _PROWS.bit_length() - 1


def _pack_body(t0, t1, t2, t3, o0, o1, o2, o3):
    ident = jnp.eye(_PW, dtype=jnp.float32)
    valid = _V - pl.program_id(0) * _PBLK
    col_ok = lax.broadcasted_iota(jnp.int32, (_D, _PBLK), 1) < valid
    for t, o in ((t0, o0), (t1, o1), (t2, o2), (t3, o3)):
        x = jnp.where(col_ok, t[...], 0.0)
        xcat = jnp.concatenate(
            [x[:, s * _PROWS:(s + 1) * _PROWS] for s in range(_PACK)],
            axis=0)
        o[...] = jax.lax.dot_general(
            xcat, ident, (((0,), (0,)), ((), ())),
            preferred_element_type=jnp.float32)


def _tc_pack(tables_t):
    grid = (_PGRID,)
    in_spec = pl.BlockSpec((_D, _PBLK), lambda i: (0, i))
    out_spec = pl.BlockSpec((_PROWS, _PW), lambda i: (i, 0))
    out = jax.ShapeDtypeStruct((_VPAD, _PW), jnp.float32)
    return pl.pallas_call(
        _pack_body,
        grid=grid,
        in_specs=[in_spec] * 4,
        out_specs=[out_spec] * 4,
        out_shape=[out] * 4,
    )(*tables_t)


def _flat_row_ids(src, dst):
    for r in range(_NCH):
        for c in range(_CH // 16):
            sl = pl.ds(c * 16, 16)
            i = src[r, sl]
            rid = lax.shift_left(
                lax.shift_right_logical(i, _BSH), _RSH) + (i & (_PROWS - 1))
            dst[r, sl] = lax.shift_left(rid, 3) + (
                lax.shift_right_logical(i, _RSH) & 7)


def _sc_gather_body(u_h, p_h, n_h, t_mfu, t_mfi, t_mlu, t_mli,
                    o_mfu, o_mfp, o_mfn, o_mlu, o_mlp, o_mln,
                    iu, ip, inn, pu, pp, pn, r0, r1, r2, r3, r4, r5, sem):
    wid = lax.axis_index("s") * _NC + lax.axis_index("c")
    pltpu.sync_copy(u_h.at[wid], iu)
    pltpu.sync_copy(p_h.at[wid], ip)
    pltpu.sync_copy(n_h.at[wid], inn)
    _flat_row_ids(iu, pu)
    _flat_row_ids(ip, pp)
    _flat_row_ids(inn, pn)
    copies = []
    for k in range(_NCH):
        sl = pl.ds(k * _CH, _CH)
        copies.append(pltpu.async_copy(t_mfu.at[pu.at[k]], r0.at[sl], sem))
        copies.append(pltpu.async_copy(t_mfi.at[pp.at[k]], r1.at[sl], sem))
        copies.append(pltpu.async_copy(t_mfi.at[pn.at[k]], r2.at[sl], sem))
        copies.append(pltpu.async_copy(t_mlu.at[pu.at[k]], r3.at[sl], sem))
        copies.append(pltpu.async_copy(t_mli.at[pp.at[k]], r4.at[sl], sem))
        copies.append(pltpu.async_copy(t_mli.at[pn.at[k]], r5.at[sl], sem))
    for c in copies:
        c.wait()
    pltpu.sync_copy(r0, o_mfu.at[wid])
    pltpu.sync_copy(r1, o_mfp.at[wid])
    pltpu.sync_copy(r2, o_mfn.at[wid])
    pltpu.sync_copy(r3, o_mlu.at[wid])
    pltpu.sync_copy(r4, o_mlp.at[wid])
    pltpu.sync_copy(r5, o_mln.at[wid])


def _sc_gather(user, pos_item, neg_item, packed_flat):
    mesh = plsc.VectorSubcoreMesh(core_axis_name="c", subcore_axis_name="s")
    row = jax.ShapeDtypeStruct((_NW, _BPW, _D), jnp.float32)
    fn = functools.partial(
        pl.kernel,
        mesh=mesh,
        out_type=[row] * 6,
        scratch_types=[pltpu.VMEM((_NCH, _CH), jnp.int32)] * 6 + [
            pltpu.VMEM((_BPW, _D), jnp.float32)] * 6 + [
            pltpu.SemaphoreType.DMA,
        ],
        compiler_params=pltpu.CompilerParams(use_tc_tiling_on_sc=False),
    )(_sc_gather_body)
    outs = fn(user.reshape(_NW, _NCH, _CH), pos_item.reshape(_NW, _NCH, _CH),
              neg_item.reshape(_NW, _NCH, _CH), *packed_flat)
    return tuple(o.reshape(_B, _D) for o in outs)


_BLK = 2048


def _score_body(mfu, mfp, mfn, mlu, mlpos, mlneg,
                w0, b0, w1, b1, w2, b2, w3, b3, dw, db, out):
    f32 = jnp.float32
    mf_u = mfu[...]
    pos_mf = jax.nn.sigmoid(mf_u * mfp[...])
    neg_mf = jax.nn.sigmoid(mf_u * mfn[...])
    ml_u = mlu[...]
    x = jnp.concatenate(
        [jnp.concatenate([ml_u, mlpos[...]], axis=1),
         jnp.concatenate([ml_u, mlneg[...]], axis=1)], axis=0)
    for w, b in ((w0, b0), (w1, b1), (w2, b2), (w3, b3)):
        x = jnp.maximum(jnp.dot(x, w[...], preferred_element_type=f32)
                        + b[...], 0.0)
    dwv = dw[...]
    dbv = db[...]
    pos_vec = jnp.concatenate([pos_mf, x[:_BLK]], axis=1)
    neg_vec = jnp.concatenate([neg_mf, x[_BLK:]], axis=1)
    pos_s = jnp.dot(pos_vec, dwv, preferred_element_type=f32) + dbv
    neg_s = jnp.dot(neg_vec, dwv, preferred_element_type=f32) + dbv
    out[...] = jnp.concatenate([pos_s, neg_s], axis=1)


def _tc_score(gathered, w0, b0, w1, b1, w2, b2, w3, b3, dense_w, dense_b):
    grid = _B // _BLK
    row_spec = pl.BlockSpec((_BLK, _D), lambda i: (i, 0))
    full = lambda a: pl.BlockSpec(a.shape, lambda i: (0,) * a.ndim)
    weights = (w0, b0.reshape(1, -1), w1, b1.reshape(1, -1),
               w2, b2.reshape(1, -1), w3, b3.reshape(1, -1),
               dense_w, dense_b.reshape(1, 1))
    return pl.pallas_call(
        _score_body,
        grid=(grid,),
        in_specs=[row_spec] * 6 + [full(w) for w in weights],
        out_specs=pl.BlockSpec((_BLK, 2), lambda i: (i, 0)),
        out_shape=jax.ShapeDtypeStruct((_B, 2), jnp.float32),
    )(*gathered, *weights)


def kernel(user, pos_item, neg_item, mf_user_table, mf_item_table,
           mlp_user_table, mlp_item_table, W0, b0, W1, b1, W2, b2, W3, b3,
           dense_W, dense_b):
    user = user.reshape(-1).astype(jnp.int32)
    pos_item = pos_item.reshape(-1).astype(jnp.int32)
    neg_item = neg_item.reshape(-1).astype(jnp.int32)
    packed = _tc_pack((mf_user_table.T, mf_item_table.T,
                       mlp_user_table.T, mlp_item_table.T))
    packed_flat = tuple(p.reshape(_VROWS, _D) for p in packed)
    gathered = _sc_gather(user, pos_item, neg_item, packed_flat)
    return _tc_score(gathered, W0, b0, W1, b1, W2, b2, W3, b3,
                     dense_W, dense_b)


# speedup vs baseline: 6.6893x; 1.0963x over previous
import functools

import jax
import jax.numpy as jnp
from jax import lax
from jax.experimental import pallas as pl
from jax.experimental.pallas import tpu as pltpu
from jax.experimental.pallas import tpu_sc as plsc

_B, _V, _D = 16384, 1000000, 16
_PACK = 8
_PW = _PACK * _D
_NC, _NS = 2, 16
_NW = _NC * _NS
_BPW = _B // _NW
_CH = 128
_NCH = _BPW // _CH

_PBLK = 16384
_PROWS = _PBLK // _PACK
_PGRID = -(-_V // _PBLK)
_VPAD = _PGRID * _PROWS
_VROWS = _VPAD * _PACK
_BSH = _PBLK.bit_length() - 1
_RSH = _PROWS.bit_length() - 1


def _pack_body(t0, t1, t2, t3, o0, o1, o2, o3):
    ident = jnp.eye(_PW, dtype=jnp.float32)
    valid = _V - pl.program_id(0) * _PBLK
    col_ok = lax.broadcasted_iota(jnp.int32, (_D, _PBLK), 1) < valid
    for t, o in ((t0, o0), (t1, o1), (t2, o2), (t3, o3)):
        x = jnp.where(col_ok, t[...], 0.0)
        xcat = jnp.concatenate(
            [x[:, s * _PROWS:(s + 1) * _PROWS] for s in range(_PACK)],
            axis=0)
        o[...] = jax.lax.dot_general(
            xcat, ident, (((0,), (0,)), ((), ())),
            preferred_element_type=jnp.float32)


def _tc_pack(tables_t):
    grid = (_PGRID,)
    in_spec = pl.BlockSpec((_D, _PBLK), lambda i: (0, i))
    out_spec = pl.BlockSpec((_PROWS, _PW), lambda i: (i, 0))
    out = jax.ShapeDtypeStruct((_VPAD, _PW), jnp.float32)
    return pl.pallas_call(
        _pack_body,
        grid=grid,
        in_specs=[in_spec] * 4,
        out_specs=[out_spec] * 4,
        out_shape=[out] * 4,
    )(*tables_t)


def _flat_row_ids(src, dst):
    for r in range(_NCH):
        for c in range(_CH // 16):
            sl = pl.ds(c * 16, 16)
            i = src[r, sl]
            rid = lax.shift_left(
                lax.shift_right_logical(i, _BSH), _RSH) + (i & (_PROWS - 1))
            dst[r, sl] = lax.shift_left(rid, 3) + (
                lax.shift_right_logical(i, _RSH) & 7)


def _sc_gather_body(u_h, p_h, n_h, t_mfu, t_mfi, t_mlu, t_mli, o_all,
                    iu, ip, inn, pu, pp, pn, r0, r1, r2, r3, r4, r5, sem):
    wid = lax.axis_index("s") * _NC + lax.axis_index("c")
    pltpu.sync_copy(u_h.at[wid], iu)
    pltpu.sync_copy(p_h.at[wid], ip)
    pltpu.sync_copy(n_h.at[wid], inn)
    _flat_row_ids(iu, pu)
    _flat_row_ids(ip, pp)
    _flat_row_ids(inn, pn)
    bufs = (r0, r1, r2, r3, r4, r5)
    copies = []
    for k in range(_NCH):
        sl = pl.ds(k * _CH, _CH)
        copies.append(pltpu.async_copy(t_mfu.at[pu.at[k]], r0.at[sl], sem))
        copies.append(pltpu.async_copy(t_mfi.at[pp.at[k]], r1.at[sl], sem))
        copies.append(pltpu.async_copy(t_mfi.at[pn.at[k]], r2.at[sl], sem))
        copies.append(pltpu.async_copy(t_mlu.at[pu.at[k]], r3.at[sl], sem))
        copies.append(pltpu.async_copy(t_mli.at[pp.at[k]], r4.at[sl], sem))
        copies.append(pltpu.async_copy(t_mli.at[pn.at[k]], r5.at[sl], sem))
    for c in copies:
        c.wait()
    for t, r in enumerate(bufs):
        pltpu.sync_copy(r, o_all.at[wid, :, pl.ds(t * _D, _D)])


def _sc_gather(user, pos_item, neg_item, packed_flat):
    mesh = plsc.VectorSubcoreMesh(core_axis_name="c", subcore_axis_name="s")
    fn = functools.partial(
        pl.kernel,
        mesh=mesh,
        out_type=jax.ShapeDtypeStruct((_NW, _BPW, 6 * _D), jnp.float32),
        scratch_types=[pltpu.VMEM((_NCH, _CH), jnp.int32)] * 6 + [
            pltpu.VMEM((_BPW, _D), jnp.float32)] * 6 + [
            pltpu.SemaphoreType.DMA,
        ],
        compiler_params=pltpu.CompilerParams(use_tc_tiling_on_sc=False),
    )(_sc_gather_body)
    out = fn(user.reshape(_NW, _NCH, _CH), pos_item.reshape(_NW, _NCH, _CH),
             neg_item.reshape(_NW, _NCH, _CH), *packed_flat)
    return out.reshape(_B, 6 * _D)


_BLK = 2048


def _score_body(gall, w0, b0, w1, b1, w2, b2, w3, b3, dw, db, out):
    f32 = jnp.float32
    g = gall[...]
    mf_u = g[:, 0:_D]
    pos_mf = jax.nn.sigmoid(mf_u * g[:, _D:2 * _D])
    neg_mf = jax.nn.sigmoid(mf_u * g[:, 2 * _D:3 * _D])
    x = jnp.concatenate(
        [g[:, 3 * _D:5 * _D],
         jnp.concatenate([g[:, 3 * _D:4 * _D], g[:, 5 * _D:6 * _D]],
                         axis=1)], axis=0)
    for w, b in ((w0, b0), (w1, b1), (w2, b2), (w3, b3)):
        x = jnp.maximum(jnp.dot(x, w[...], preferred_element_type=f32)
                        + b[...], 0.0)
    dwv = dw[...]
    dbv = db[...]
    pos_vec = jnp.concatenate([pos_mf, x[:_BLK]], axis=1)
    neg_vec = jnp.concatenate([neg_mf, x[_BLK:]], axis=1)
    pos_s = jnp.dot(pos_vec, dwv, preferred_element_type=f32) + dbv
    neg_s = jnp.dot(neg_vec, dwv, preferred_element_type=f32) + dbv
    out[...] = jnp.concatenate([pos_s, neg_s], axis=1)


def _tc_score(gathered, w0, b0, w1, b1, w2, b2, w3, b3, dense_w, dense_b):
    grid = _B // _BLK
    row_spec = pl.BlockSpec((_BLK, 6 * _D), lambda i: (i, 0))
    full = lambda a: pl.BlockSpec(a.shape, lambda i: (0,) * a.ndim)
    weights = (w0, b0.reshape(1, -1), w1, b1.reshape(1, -1),
               w2, b2.reshape(1, -1), w3, b3.reshape(1, -1),
               dense_w, dense_b.reshape(1, 1))
    return pl.pallas_call(
        _score_body,
        grid=(grid,),
        in_specs=[row_spec] + [full(w) for w in weights],
        out_specs=pl.BlockSpec((_BLK, 2), lambda i: (i, 0)),
        out_shape=jax.ShapeDtypeStruct((_B, 2), jnp.float32),
    )(gathered, *weights)


def kernel(user, pos_item, neg_item, mf_user_table, mf_item_table,
           mlp_user_table, mlp_item_table, W0, b0, W1, b1, W2, b2, W3, b3,
           dense_W, dense_b):
    user = user.reshape(-1).astype(jnp.int32)
    pos_item = pos_item.reshape(-1).astype(jnp.int32)
    neg_item = neg_item.reshape(-1).astype(jnp.int32)
    packed = _tc_pack((mf_user_table.T, mf_item_table.T,
                       mlp_user_table.T, mlp_item_table.T))
    packed_flat = tuple(p.reshape(_VROWS, _D) for p in packed)
    gathered = _sc_gather(user, pos_item, neg_item, packed_flat)
    return _tc_score(gathered, W0, b0, W1, b1, W2, b2, W3, b3,
                     dense_W, dense_b)


# speedup vs baseline: 6.7814x; 1.0138x over previous
import functools

import jax
import jax.numpy as jnp
from jax import lax
from jax.experimental import pallas as pl
from jax.experimental.pallas import tpu as pltpu
from jax.experimental.pallas import tpu_sc as plsc

_B, _V, _D = 16384, 1000000, 16
_PACK = 8
_PW = _PACK * _D
_NC, _NS = 2, 16
_NW = _NC * _NS
_BPW = _B // _NW
_CH = 128
_NCH = _BPW // _CH

_PBLK = 32768
_PROWS = _PBLK // _PACK
_PGRID = -(-_V // _PBLK)
_VPAD = _PGRID * _PROWS
_VROWS = _VPAD * _PACK
_BSH = _PBLK.bit_length() - 1
_RSH = _PROWS.bit_length() - 1


def _pack_body(t0, t1, t2, t3, o0, o1, o2, o3):
    ident = jnp.eye(_PW, dtype=jnp.float32)
    valid = _V - pl.program_id(0) * _PBLK
    col_ok = lax.broadcasted_iota(jnp.int32, (_D, _PBLK), 1) < valid
    for t, o in ((t0, o0), (t1, o1), (t2, o2), (t3, o3)):
        x = jnp.where(col_ok, t[...], 0.0)
        xcat = jnp.concatenate(
            [x[:, s * _PROWS:(s + 1) * _PROWS] for s in range(_PACK)],
            axis=0)
        o[...] = jax.lax.dot_general(
            xcat, ident, (((0,), (0,)), ((), ())),
            preferred_element_type=jnp.float32)


def _tc_pack(tables_t):
    grid = (_PGRID,)
    in_spec = pl.BlockSpec((_D, _PBLK), lambda i: (0, i))
    out_spec = pl.BlockSpec((_PROWS, _PW), lambda i: (i, 0))
    out = jax.ShapeDtypeStruct((_VPAD, _PW), jnp.float32)
    return pl.pallas_call(
        _pack_body,
        grid=grid,
        in_specs=[in_spec] * 4,
        out_specs=[out_spec] * 4,
        out_shape=[out] * 4,
    )(*tables_t)


def _flat_row_ids(src, dst):
    for r in range(_NCH):
        for c in range(_CH // 16):
            sl = pl.ds(c * 16, 16)
            i = src[r, sl]
            rid = lax.shift_left(
                lax.shift_right_logical(i, _BSH), _RSH) + (i & (_PROWS - 1))
            dst[r, sl] = lax.shift_left(rid, 3) + (
                lax.shift_right_logical(i, _RSH) & 7)


def _sc_gather_body(u_h, p_h, n_h, t_mfu, t_mfi, t_mlu, t_mli, o_all,
                    iu, ip, inn, pu, pp, pn, r0, r1, r2, r3, r4, r5, sem):
    wid = lax.axis_index("s") * _NC + lax.axis_index("c")
    pltpu.sync_copy(u_h.at[wid], iu)
    pltpu.sync_copy(p_h.at[wid], ip)
    pltpu.sync_copy(n_h.at[wid], inn)
    _flat_row_ids(iu, pu)
    _flat_row_ids(ip, pp)
    _flat_row_ids(inn, pn)
    bufs = (r0, r1, r2, r3, r4, r5)
    copies = []
    for k in range(_NCH):
        sl = pl.ds(k * _CH, _CH)
        copies.append(pltpu.async_copy(t_mfu.at[pu.at[k]], r0.at[sl], sem))
        copies.append(pltpu.async_copy(t_mfi.at[pp.at[k]], r1.at[sl], sem))
        copies.append(pltpu.async_copy(t_mfi.at[pn.at[k]], r2.at[sl], sem))
        copies.append(pltpu.async_copy(t_mlu.at[pu.at[k]], r3.at[sl], sem))
        copies.append(pltpu.async_copy(t_mli.at[pp.at[k]], r4.at[sl], sem))
        copies.append(pltpu.async_copy(t_mli.at[pn.at[k]], r5.at[sl], sem))
    for c in copies:
        c.wait()
    for t, r in enumerate(bufs):
        pltpu.sync_copy(r, o_all.at[wid, :, pl.ds(t * _D, _D)])


def _sc_gather(user, pos_item, neg_item, packed_flat):
    mesh = plsc.VectorSubcoreMesh(core_axis_name="c", subcore_axis_name="s")
    fn = functools.partial(
        pl.kernel,
        mesh=mesh,
        out_type=jax.ShapeDtypeStruct((_NW, _BPW, 6 * _D), jnp.float32),
        scratch_types=[pltpu.VMEM((_NCH, _CH), jnp.int32)] * 6 + [
            pltpu.VMEM((_BPW, _D), jnp.float32)] * 6 + [
            pltpu.SemaphoreType.DMA,
        ],
        compiler_params=pltpu.CompilerParams(use_tc_tiling_on_sc=False),
    )(_sc_gather_body)
    out = fn(user.reshape(_NW, _NCH, _CH), pos_item.reshape(_NW, _NCH, _CH),
             neg_item.reshape(_NW, _NCH, _CH), *packed_flat)
    return out.reshape(_B, 6 * _D)


_BLK = 2048


def _score_body(gall, w0, b0, w1, b1, w2, b2, w3, b3, dw, db, out):
    f32 = jnp.float32
    g = gall[...]
    mf_u = g[:, 0:_D]
    pos_mf = jax.nn.sigmoid(mf_u * g[:, _D:2 * _D])
    neg_mf = jax.nn.sigmoid(mf_u * g[:, 2 * _D:3 * _D])
    x = jnp.concatenate(
        [g[:, 3 * _D:5 * _D],
         jnp.concatenate([g[:, 3 * _D:4 * _D], g[:, 5 * _D:6 * _D]],
                         axis=1)], axis=0)
    for w, b in ((w0, b0), (w1, b1), (w2, b2), (w3, b3)):
        x = jnp.maximum(jnp.dot(x, w[...], preferred_element_type=f32)
                        + b[...], 0.0)
    dwv = dw[...]
    dbv = db[...]
    pos_vec = jnp.concatenate([pos_mf, x[:_BLK]], axis=1)
    neg_vec = jnp.concatenate([neg_mf, x[_BLK:]], axis=1)
    pos_s = jnp.dot(pos_vec, dwv, preferred_element_type=f32) + dbv
    neg_s = jnp.dot(neg_vec, dwv, preferred_element_type=f32) + dbv
    out[...] = jnp.concatenate([pos_s, neg_s], axis=1)


def _tc_score(gathered, w0, b0, w1, b1, w2, b2, w3, b3, dense_w, dense_b):
    grid = _B // _BLK
    row_spec = pl.BlockSpec((_BLK, 6 * _D), lambda i: (i, 0))
    full = lambda a: pl.BlockSpec(a.shape, lambda i: (0,) * a.ndim)
    weights = (w0, b0.reshape(1, -1), w1, b1.reshape(1, -1),
               w2, b2.reshape(1, -1), w3, b3.reshape(1, -1),
               dense_w, dense_b.reshape(1, 1))
    return pl.pallas_call(
        _score_body,
        grid=(grid,),
        in_specs=[row_spec] + [full(w) for w in weights],
        out_specs=pl.BlockSpec((_BLK, 2), lambda i: (i, 0)),
        out_shape=jax.ShapeDtypeStruct((_B, 2), jnp.float32),
    )(gathered, *weights)


def kernel(user, pos_item, neg_item, mf_user_table, mf_item_table,
           mlp_user_table, mlp_item_table, W0, b0, W1, b1, W2, b2, W3, b3,
           dense_W, dense_b):
    user = user.reshape(-1).astype(jnp.int32)
    pos_item = pos_item.reshape(-1).astype(jnp.int32)
    neg_item = neg_item.reshape(-1).astype(jnp.int32)
    packed = _tc_pack((mf_user_table.T, mf_item_table.T,
                       mlp_user_table.T, mlp_item_table.T))
    packed_flat = tuple(p.reshape(_VROWS, _D) for p in packed)
    gathered = _sc_gather(user, pos_item, neg_item, packed_flat)
    return _tc_score(gathered, W0, b0, W1, b1, W2, b2, W3, b3,
                     dense_W, dense_b)
